# baseline (speedup 1.0000x reference)
import functools

import jax
import jax.numpy as jnp
from jax import lax
from jax.experimental import pallas as pl
from jax.experimental.pallas import tpu as pltpu
from jax.experimental.pallas import tpu_sc as plsc

N = 10000
NPAD = 10240
E = 320000
F = 256
NH = 4
NW = 32
NPT = NPAD // NW
CAP = 16384
ECH = 6400
NECH = E // ECH
CCH = 128
GCH = 64
FE = 384
RB = 1280
GB = NPAD // RB
MASK14 = (1 << 14) - 1

_mesh = plsc.VectorSubcoreMesh(core_axis_name="c", subcore_axis_name="s")


def _wid():
    return lax.axis_index("s") * 2 + lax.axis_index("c")


@functools.partial(
    pl.kernel,
    mesh=_mesh,
    compiler_params=pltpu.CompilerParams(needs_layout_passes=False),
    out_type=[
        jax.ShapeDtypeStruct((2, NW, CAP), jnp.int32),
        jax.ShapeDtypeStruct((2, NW, 128), jnp.int32),
    ],
    scratch_types=[
        pltpu.VMEM((ECH,), jnp.int32),
        pltpu.VMEM((ECH,), jnp.int32),
        pltpu.VMEM((CAP + 64,), jnp.int32),
        pltpu.VMEM((128,), jnp.int32),
    ],
)
def _bin_edges(ei0_ref, ei1_ref, bins_ref, cnts_ref, src_st, dst_st, outb, cvec):
    wid = _wid()
    lo = wid * NPT
    hi = lo + NPT
    for g in range(2):
        ei_ref = (ei0_ref, ei1_ref)[g]

        def chunk(c, cnt):
            off = pl.multiple_of(c * ECH, ECH)
            pltpu.sync_copy(ei_ref.at[0, pl.ds(off, ECH)], src_st)
            pltpu.sync_copy(ei_ref.at[1, pl.ds(off, ECH)], dst_st)

            def grp(k, cnt):
                for u in range(4):
                    o = k * 64 + u * 16
                    s = src_st[pl.ds(o, 16)]
                    d = dst_st[pl.ds(o, 16)]
                    msk = (d >= lo) & (d < hi)
                    packed = s | ((d - lo) << 14)
                    cs = plsc.cumsum(msk.astype(jnp.int32))
                    pos = jnp.where(msk, cnt - 1 + cs, CAP + 48)
                    plsc.store_scatter(outb, [pos], packed)
                    cnt = cnt + plsc.all_reduce_population_count(msk)
                return cnt

            return lax.fori_loop(0, ECH // 64, grp, cnt)

        cnt = lax.fori_loop(0, NECH, chunk, jnp.zeros((16,), jnp.int32))
        pltpu.sync_copy(outb.at[pl.ds(0, CAP)], bins_ref.at[g, wid, :])
        for q in range(8):
            cvec[pl.ds(16 * q, 16)] = cnt
        pltpu.sync_copy(cvec, cnts_ref.at[g, wid, :])


def _make_conv(g):
    @functools.partial(
        pl.kernel,
        mesh=_mesh,
        compiler_params=pltpu.CompilerParams(needs_layout_passes=False),
        out_type=[
            jax.ShapeDtypeStruct((NPAD, F), jnp.float32),
            jax.ShapeDtypeStruct((NPAD * 16,), jnp.float32),
        ],
        scratch_types=[
            pltpu.VMEM((NPT, F), jnp.float32),
            pltpu.VMEM((NPT * 16,), jnp.float32),
            pltpu.VMEM((NPT * 16,), jnp.float32),
            pltpu.VMEM((CCH,), jnp.int32),
            pltpu.VMEM((GCH,), jnp.int32),
            pltpu.VMEM((GCH + 16,), jnp.int32),
            pltpu.VMEM((GCH, FE), jnp.float32),
            pltpu.VMEM((128,), jnp.float32),
            pltpu.SemaphoreType.DMA,
        ],
    )
    def _conv(bins_ref, cnts_ref, h_ref, er_ref, m16_ref,
              acc_hbm, den_hbm,
              acc, den, erl, pk, srcb, dstb, h_st, m_v,
              sem1):
        wid = _wid()
        base = wid * NPT
        pltpu.sync_copy(er_ref.at[pl.ds(base * 16, NPT * 16)], erl)
        pltpu.sync_copy(cnts_ref.at[g, wid, :], pk.at[pl.ds(0, 128)])
        count = pk[pl.ds(0, 16)][0]
        pltpu.sync_copy(m16_ref, m_v)
        mv = m_v[pl.ds(0, 16)]

        def zero(i, _):
            for j in range(F // 16):
                acc[i, pl.ds(16 * j, 16)] = jnp.zeros((16,), jnp.float32)
            den[pl.ds(i * 16, 16)] = jnp.zeros((16,), jnp.float32)
            return 0

        lax.fori_loop(0, NPT, zero, 0)

        nch = (count + CCH - 1) // CCH

        def chunk(c, _):
            off = pl.multiple_of(c * CCH, CCH)
            pltpu.sync_copy(bins_ref.at[g, wid, pl.ds(off, CCH)], pk)
            for half in range(2):
                hoff = off + GCH * half
                for u in range(GCH // 16):
                    v = pk[pl.ds(GCH * half + 16 * u, 16)]
                    eidx = hoff + 16 * u + lax.iota(jnp.int32, 16)
                    v = jnp.where(eidx < count, v, 0)
                    srcb[pl.ds(16 * u, 16)] = v & MASK14
                    dstb[pl.ds(16 * u, 16)] = lax.shift_right_logical(v, 14)
                pltpu.async_copy(h_ref.at[srcb], h_st, sem1).wait()
                nv = jnp.minimum(GCH, count - hoff)

                def edge(e, _):
                    dl = dstb[pl.ds(e, 16)][0]
                    ev = h_st[e, pl.ds(F, 16)] + erl[pl.ds(dl * 16, 16)]
                    ev = jnp.maximum(ev, 0.2 * ev) - mv
                    eev = jnp.exp(ev)
                    plsc.addupdate(den.at[pl.ds(dl * 16, 16)], eev)
                    ss = (eev[0], eev[1], eev[2], eev[3])
                    for j in range(F // 16):
                        plsc.addupdate(acc.at[dl, pl.ds(16 * j, 16)],
                                       h_st[e, pl.ds(16 * j, 16)] * ss[j // 4])
                    return 0

                lax.fori_loop(0, nv, edge, 0)
            return 0

        lax.fori_loop(0, nch, chunk, 0)
        pltpu.sync_copy(acc, acc_hbm.at[pl.ds(base, NPT), :])
        pltpu.sync_copy(den, den_hbm.at[pl.ds(base * 16, NPT * 16)])

    return _conv


_conv_g0 = _make_conv(0)
_conv_g1 = _make_conv(1)


@functools.partial(
    pl.kernel,
    mesh=_mesh,
    compiler_params=pltpu.CompilerParams(needs_layout_passes=False),
    out_type=jax.ShapeDtypeStruct((NPAD, 16), jnp.float32),
    scratch_types=[
        pltpu.VMEM((16, F), jnp.float32),
        pltpu.VMEM((9 * 384,), jnp.int32),
        pltpu.VMEM((128,), jnp.int32),
        pltpu.VMEM((16,), jnp.int32),
        pltpu.VMEM((128, F), jnp.float32),
        pltpu.VMEM((16, F), jnp.float32),
        pltpu.VMEM((16, 16), jnp.float32),
        pltpu.SemaphoreType.DMA,
        pltpu.SemaphoreType.DMA,
    ],
)
def _loss_gather(rep_ref, idx_ref, af_hbm,
                 own_st, idxall, idxb1, idxb2, g1, g2, afst, sem1, sem2):
    wid = _wid()
    base = wid * NPT
    lane = lax.iota(jnp.int32, 16)
    for k in range(9):
        pltpu.sync_copy(idx_ref.at[k, wid, :], idxall.at[pl.ds(384 * k, 384)])

    def sub(s, _):
        nb = pl.multiple_of(base + s * 16, 16)
        pltpu.sync_copy(rep_ref.at[pl.ds(nb, 16), :], own_st)
        for k in range(8):
            idxb1[pl.ds(16 * k, 16)] = idxall[pl.ds(384 * k + 16 * s, 16)]
        idxb2[...] = idxall[pl.ds(384 * 8 + 16 * s, 16)]
        cp1 = pltpu.async_copy(rep_ref.at[idxb1], g1, sem1)
        cp2 = pltpu.async_copy(rep_ref.at[idxb2], g2, sem2)
        cp1.wait()
        cp2.wait()

        def node(i, _):
            own = [own_st[i, pl.ds(16 * j, 16)] for j in range(F // 16)]
            row = jnp.zeros((16,), jnp.float32)
            for k in range(9):
                if k < 8:
                    accv = own[0] * g1[16 * k + i, pl.ds(0, 16)]
                    for j in range(1, F // 16):
                        accv = accv + own[j] * g1[16 * k + i, pl.ds(16 * j, 16)]
                else:
                    accv = own[0] * g2[i, pl.ds(0, 16)]
                    for j in range(1, F // 16):
                        accv = accv + own[j] * g2[i, pl.ds(16 * j, 16)]
                row = jnp.where(lane == k, jnp.sum(accv), row)
            afst[i, :] = row
            return 0

        lax.fori_loop(0, 16, node, 0)
        pltpu.sync_copy(afst, af_hbm.at[pl.ds(nb, 16), :])
        return 0

    lax.fori_loop(0, NPT // 16, sub, 0)


def _proj0_body(x_ref, W0_ref, W1_ref, A0_ref, A1_ref,
                h0_ref, h1_ref, mx_ref):
    _proj_core(x_ref[...], (W0_ref, W1_ref), (A0_ref, A1_ref),
               (h0_ref, h1_ref), mx_ref)


def _proj1_body(x0_ref, x1_ref, beta_ref, W0_ref, W1_ref, A0_ref, A1_ref,
                h0_ref, h1_ref, mx_ref):
    x = beta_ref[0, 0] * x0_ref[...] + beta_ref[0, 1] * x1_ref[...]
    _proj_core(x, (W0_ref, W1_ref), (A0_ref, A1_ref),
               (h0_ref, h1_ref), mx_ref)


def _proj_core(x, W_refs, A_refs, h_refs, mx_ref):
    mxs = []
    for m in range(2):
        h = jnp.dot(x, W_refs[m][...], preferred_element_type=jnp.float32)
        ee = jnp.dot(h, A_refs[m][...], preferred_element_type=jnp.float32)
        h_refs[m][:, 0:F] = h
        h_refs[m][:, F:F + 32] = ee
        mxs.append(jnp.max(ee, axis=0).reshape(1, 1, 32))
    mx_ref[...] = jnp.concatenate(mxs, axis=2)


def _combine_body(a0_ref, d0_ref, h0_ref,
                  a1_ref, d1_ref, h1_ref,
                  m32_ref, X16_ref, Ws_ref, bs_ref, qs_ref,
                  z0_ref, z1_ref, wp_ref):
    gi = pl.program_id(0)
    rid = gi * RB + lax.broadcasted_iota(jnp.int32, (RB, 1), 0)
    msk = (rid < N).astype(jnp.float32)
    X16 = X16_ref[...]
    ws = []
    for m in range(2):
        h384 = (h0_ref, h1_ref)[m][...]
        h = h384[:, 0:F]
        el = h384[:, F:F + 16]
        er = h384[:, F + 16:F + 32]
        m16 = m32_ref[:, 16 * m:16 * m + 16]
        es = el + er
        es = jnp.maximum(es, 0.2 * es) - m16
        ees = jnp.exp(es)
        den = (d0_ref, d1_ref)[m][...] + ees
        numer = (a0_ref, a1_ref)[m][...] + h * jnp.dot(
            ees, X16, preferred_element_type=jnp.float32)
        z = numer / jnp.dot(den, X16, preferred_element_type=jnp.float32)
        z = jnp.where(z > 0, z, jnp.exp(z) - 1.0)
        ((z0_ref, z1_ref)[m])[...] = z
        t = jnp.tanh(jnp.dot(z, Ws_ref[...],
                             preferred_element_type=jnp.float32) + bs_ref[...])
        wv = jnp.sum(t * qs_ref[...], axis=1, keepdims=True)
        ws.append(jnp.sum(wv * msk).reshape(1, 1, 1))
    wp_ref[...] = jnp.concatenate(
        [ws[0], ws[1], jnp.zeros((1, 1, 126), jnp.float32)], axis=2)


def _rep_body(x0_ref, x1_ref, beta_ref, W_ref, rep_ref):
    x = beta_ref[0, 0] * x0_ref[...] + beta_ref[0, 1] * x1_ref[...]
    rep_ref[...] = jnp.dot(x, W_ref[...], preferred_element_type=jnp.float32)


def _loss_body(af_ref, out_ref):
    af = af_ref[...]
    rid = lax.broadcasted_iota(jnp.int32, (NPAD, 1), 0)
    msk = (rid < N).astype(jnp.float32)

    def sp(x):
        return jnp.maximum(x, 0.0) + jnp.log(1.0 + jnp.exp(-jnp.abs(x)))

    s_pos = jnp.sum(sp(-af[:, 0:4]) * msk)
    s_neg = jnp.sum(sp(af[:, 4:9]) * msk)
    out_ref[...] = jnp.reshape((s_pos + s_neg) / N, (1, 1))


def _f32(*shape):
    return jax.ShapeDtypeStruct(shape, jnp.float32)


def _row_spec(shape):
    return pl.BlockSpec(shape, lambda i: (i,) + (0,) * (len(shape) - 1))


def _fix_spec(shape):
    return pl.BlockSpec(shape, lambda i: (0,) * len(shape))


_SMEM_SPEC = pl.BlockSpec(memory_space=pltpu.SMEM)


def _proj_call(x_list, beta, W0, W1, A0, A1, din):
    outs = [_f32(NPAD, FE), _f32(NPAD, FE), _f32(GB, 1, 64)]
    out_specs = [_row_spec((RB, FE)), _row_spec((RB, FE)),
                 _row_spec((1, 1, 64))]
    w_specs = [_fix_spec((din, F)), _fix_spec((din, F)),
               _fix_spec((F, 32)), _fix_spec((F, 32))]
    if beta is None:
        return pl.pallas_call(
            _proj0_body, grid=(GB,),
            in_specs=[_row_spec((RB, din))] + w_specs,
            out_specs=out_specs, out_shape=outs,
        )(x_list[0], W0, W1, A0, A1)
    return pl.pallas_call(
        _proj1_body, grid=(GB,),
        in_specs=[_row_spec((RB, din)), _row_spec((RB, din)), _SMEM_SPEC]
        + w_specs,
        out_specs=out_specs, out_shape=outs,
    )(x_list[0], x_list[1], beta, W0, W1, A0, A1)


def _combine_call(a0, d0, h0, a1, d1, h1, m32, X16, Ws, bs, qs):
    return pl.pallas_call(
        _combine_body, grid=(GB,),
        in_specs=[
            _row_spec((RB, F)), _row_spec((RB, 16)), _row_spec((RB, FE)),
            _row_spec((RB, F)), _row_spec((RB, 16)), _row_spec((RB, FE)),
            _fix_spec((1, 32)), _fix_spec((16, F)), _fix_spec((F, 128)),
            _fix_spec((1, 128)), _fix_spec((1, 128)),
        ],
        out_specs=[_row_spec((RB, F)), _row_spec((RB, F)),
                   _row_spec((1, 1, 128))],
        out_shape=[_f32(NPAD, F), _f32(NPAD, F), _f32(GB, 1, 128)],
    )(a0, d0, h0, a1, d1, h1, m32, X16, Ws, bs, qs)


def _maxes(mx, m):
    blk = mx[:, 0, 32 * m:32 * m + 32]
    return jnp.max(blk[:, 0:16], axis=0) + jnp.max(blk[:, 16:32], axis=0)


def kernel(feats, mps_edge_index_0, mps_edge_index_1, train_edge_false_index,
           train_edge_f_index, train_edge_index, train_edge_ulu_index,
           train_edge_ullu_index, W_l0_m0, al_l0_m0, ar_l0_m0, W_l0_m1,
           al_l0_m1, ar_l0_m1, Ws_l0, bs_l0, qs_l0, W_l1_m0, al_l1_m0,
           ar_l1_m0, W_l1_m1, al_l1_m1, ar_l1_m1, Ws_l1, bs_l1, qs_l1, W_ed):
    f32 = jnp.float32
    xp = jnp.pad(feats, ((0, NPAD - N), (0, 0)))
    onehot = (jnp.arange(F)[:, None] // 64 == jnp.arange(16)[None, :]).astype(f32)

    def _A(al, ar):
        return jnp.concatenate([onehot * al.reshape(F)[:, None],
                                onehot * ar.reshape(F)[:, None]], axis=1)

    X16 = (jnp.arange(F)[None, :] // 64 == jnp.arange(16)[:, None]).astype(f32)
    A00, A01 = _A(al_l0_m0, ar_l0_m0), _A(al_l0_m1, ar_l0_m1)
    A10, A11 = _A(al_l1_m0, ar_l1_m0), _A(al_l1_m1, ar_l1_m1)

    bins, cnts = _bin_edges(mps_edge_index_0, mps_edge_index_1)

    zpair = None
    beta2d = None
    for l in range(2):
        W0, W1 = (W_l0_m0, W_l0_m1) if l == 0 else (W_l1_m0, W_l1_m1)
        A0, A1 = (A00, A01) if l == 0 else (A10, A11)
        Ws, bs, qs = (Ws_l0, bs_l0, qs_l0) if l == 0 else (Ws_l1, bs_l1, qs_l1)
        din = 128 if l == 0 else F
        x_list = [xp] if l == 0 else zpair
        h0, h1, mx = _proj_call(x_list, beta2d, W0, W1, A0, A1, din)
        M0 = _maxes(mx, 0)
        M1 = _maxes(mx, 1)
        er0f = h0[:, F + 16:F + 32].reshape(-1)
        er1f = h1[:, F + 16:F + 32].reshape(-1)
        acc0, den0 = _conv_g0(bins, cnts, h0, er0f, jnp.pad(M0, (0, 112)))
        acc1, den1 = _conv_g1(bins, cnts, h1, er1f, jnp.pad(M1, (0, 112)))
        den0 = den0.reshape(NPAD, 16)
        den1 = den1.reshape(NPAD, 16)
        m32 = jnp.concatenate([M0, M1]).reshape(1, 32)
        z0, z1, wp = _combine_call(acc0, den0, h0, acc1, den1, h1,
                                   m32, X16, Ws, bs.reshape(1, 128),
                                   qs.reshape(1, 128))
        beta = jax.nn.softmax(jnp.sum(wp[:, 0, 0:2], axis=0) / N)
        beta2d = beta.reshape(1, 2)
        zpair = [z0, z1]

    rep = pl.pallas_call(
        _rep_body, grid=(GB,),
        in_specs=[_row_spec((RB, F)), _row_spec((RB, F)), _SMEM_SPEC,
                  _fix_spec((F, F))],
        out_specs=_row_spec((RB, F)),
        out_shape=_f32(NPAD, F),
    )(zpair[0], zpair[1], beta2d, W_ed)

    idxs = jnp.concatenate([
        train_edge_f_index[None, :], train_edge_index[None, :],
        train_edge_ulu_index[None, :], train_edge_ullu_index[None, :],
        train_edge_false_index.T,
    ], axis=0)
    idx9 = jnp.pad(idxs, ((0, 0), (0, NPAD - N))).reshape(9, NW, NPT)
    idx9 = jnp.pad(idx9, ((0, 0), (0, 0), (0, 384 - NPT)))

    af = _loss_gather(rep, idx9)

    loss = pl.pallas_call(
        _loss_body,
        out_shape=_f32(1, 1),
    )(af)
    return jnp.reshape(loss, ())


# speedup vs baseline: 1.0090x; 1.0090x over previous
import functools

import jax
import jax.numpy as jnp
from jax import lax
from jax.experimental import pallas as pl
from jax.experimental.pallas import tpu as pltpu
from jax.experimental.pallas import tpu_sc as plsc

N = 10000
NPAD = 10240
E = 320000
F = 256
NH = 4
NW = 32
NPT = NPAD // NW
CAP = 16384
ECH = 6400
NECH = E // ECH
CCH = 128
GCH = 64
FE = 384
RB = 1280
GB = NPAD // RB
MASK14 = (1 << 14) - 1

_mesh = plsc.VectorSubcoreMesh(core_axis_name="c", subcore_axis_name="s")


def _wid():
    return lax.axis_index("s") * 2 + lax.axis_index("c")


@functools.partial(
    pl.kernel,
    mesh=_mesh,
    compiler_params=pltpu.CompilerParams(needs_layout_passes=False),
    out_type=[
        jax.ShapeDtypeStruct((2, NW, CAP), jnp.int32),
        jax.ShapeDtypeStruct((2, NW, 128), jnp.int32),
    ],
    scratch_types=[
        pltpu.VMEM((ECH,), jnp.int32),
        pltpu.VMEM((ECH,), jnp.int32),
        pltpu.VMEM((CAP + 64,), jnp.int32),
        pltpu.VMEM((128,), jnp.int32),
    ],
)
def _bin_edges(ei0_ref, ei1_ref, bins_ref, cnts_ref, src_st, dst_st, outb, cvec):
    wid = _wid()
    lo = wid * NPT
    hi = lo + NPT
    for g in range(2):
        ei_ref = (ei0_ref, ei1_ref)[g]

        def chunk(c, cnt):
            off = pl.multiple_of(c * ECH, ECH)
            pltpu.sync_copy(ei_ref.at[0, pl.ds(off, ECH)], src_st)
            pltpu.sync_copy(ei_ref.at[1, pl.ds(off, ECH)], dst_st)

            def grp(k, cnt):
                for u in range(4):
                    o = k * 64 + u * 16
                    s = src_st[pl.ds(o, 16)]
                    d = dst_st[pl.ds(o, 16)]
                    msk = (d >= lo) & (d < hi)
                    packed = s | ((d - lo) << 14)
                    cs = plsc.cumsum(msk.astype(jnp.int32))
                    pos = jnp.where(msk, cnt - 1 + cs, CAP + 48)
                    plsc.store_scatter(outb, [pos], packed)
                    cnt = cnt + plsc.all_reduce_population_count(msk)
                return cnt

            return lax.fori_loop(0, ECH // 64, grp, cnt)

        cnt = lax.fori_loop(0, NECH, chunk, jnp.zeros((16,), jnp.int32))
        pltpu.sync_copy(outb.at[pl.ds(0, CAP)], bins_ref.at[g, wid, :])
        for q in range(8):
            cvec[pl.ds(16 * q, 16)] = cnt
        pltpu.sync_copy(cvec, cnts_ref.at[g, wid, :])


def _make_conv(g):
    @functools.partial(
        pl.kernel,
        mesh=_mesh,
        compiler_params=pltpu.CompilerParams(needs_layout_passes=False),
        out_type=[
            jax.ShapeDtypeStruct((NPAD, F), jnp.float32),
            jax.ShapeDtypeStruct((NPAD * 16,), jnp.float32),
        ],
        scratch_types=[
            pltpu.VMEM((NPT, F), jnp.float32),
            pltpu.VMEM((NPT * 16,), jnp.float32),
            pltpu.VMEM((NPT * 16,), jnp.float32),
            pltpu.VMEM((CCH,), jnp.int32),
            pltpu.VMEM((GCH,), jnp.int32),
            pltpu.VMEM((GCH + 16,), jnp.int32),
            pltpu.VMEM((GCH, FE), jnp.float32),
            pltpu.VMEM((128,), jnp.float32),
            pltpu.SemaphoreType.DMA,
        ],
    )
    def _conv(bins_ref, cnts_ref, h_ref, er_ref, m16_ref,
              acc_hbm, den_hbm,
              acc, den, erl, pk, srcb, dstb, h_st, m_v,
              sem1):
        wid = _wid()
        base = wid * NPT
        pltpu.sync_copy(er_ref.at[pl.ds(base * 16, NPT * 16)], erl)
        pltpu.sync_copy(cnts_ref.at[g, wid, :], pk.at[pl.ds(0, 128)])
        count = pk[pl.ds(0, 16)][0]
        pltpu.sync_copy(m16_ref, m_v)
        mv = m_v[pl.ds(0, 16)]

        def zero(i, _):
            for j in range(F // 16):
                acc[i, pl.ds(16 * j, 16)] = jnp.zeros((16,), jnp.float32)
            den[pl.ds(i * 16, 16)] = jnp.zeros((16,), jnp.float32)
            return 0

        lax.fori_loop(0, NPT, zero, 0)

        nch = (count + CCH - 1) // CCH

        def chunk(c, _):
            off = pl.multiple_of(c * CCH, CCH)
            pltpu.sync_copy(bins_ref.at[g, wid, pl.ds(off, CCH)], pk)
            for half in range(2):
                hoff = off + GCH * half
                for u in range(GCH // 16):
                    v = pk[pl.ds(GCH * half + 16 * u, 16)]
                    eidx = hoff + 16 * u + lax.iota(jnp.int32, 16)
                    v = jnp.where(eidx < count, v, 0)
                    srcb[pl.ds(16 * u, 16)] = v & MASK14
                    dstb[pl.ds(16 * u, 16)] = lax.shift_right_logical(v, 14)
                pltpu.async_copy(h_ref.at[srcb], h_st, sem1).wait()
                nv = jnp.maximum(0, jnp.minimum(GCH, count - hoff))

                def do_edge(e):
                    dl = dstb[pl.ds(e, 16)][0]
                    ev = h_st[e, pl.ds(F, 16)] + erl[pl.ds(dl * 16, 16)]
                    ev = jnp.maximum(ev, 0.2 * ev) - mv
                    eev = jnp.exp(ev)
                    plsc.addupdate(den.at[pl.ds(dl * 16, 16)], eev)
                    ss = (eev[0], eev[1], eev[2], eev[3])
                    for j in range(F // 16):
                        plsc.addupdate(acc.at[dl, pl.ds(16 * j, 16)],
                                       h_st[e, pl.ds(16 * j, 16)] * ss[j // 4])

                npair = nv // 2

                def edge2(p, _):
                    do_edge(2 * p)
                    do_edge(2 * p + 1)
                    return 0

                lax.fori_loop(0, npair, edge2, 0)

                @pl.when(nv - 2 * npair == 1)
                def _tail():
                    do_edge(nv - 1)
            return 0

        lax.fori_loop(0, nch, chunk, 0)
        pltpu.sync_copy(acc, acc_hbm.at[pl.ds(base, NPT), :])
        pltpu.sync_copy(den, den_hbm.at[pl.ds(base * 16, NPT * 16)])

    return _conv


_conv_g0 = _make_conv(0)
_conv_g1 = _make_conv(1)


@functools.partial(
    pl.kernel,
    mesh=_mesh,
    compiler_params=pltpu.CompilerParams(needs_layout_passes=False),
    out_type=jax.ShapeDtypeStruct((NPAD, 16), jnp.float32),
    scratch_types=[
        pltpu.VMEM((16, F), jnp.float32),
        pltpu.VMEM((9 * 384,), jnp.int32),
        pltpu.VMEM((128,), jnp.int32),
        pltpu.VMEM((16,), jnp.int32),
        pltpu.VMEM((128, F), jnp.float32),
        pltpu.VMEM((16, F), jnp.float32),
        pltpu.VMEM((16, 16), jnp.float32),
        pltpu.SemaphoreType.DMA,
        pltpu.SemaphoreType.DMA,
    ],
)
def _loss_gather(rep_ref, idx_ref, af_hbm,
                 own_st, idxall, idxb1, idxb2, g1, g2, afst, sem1, sem2):
    wid = _wid()
    base = wid * NPT
    lane = lax.iota(jnp.int32, 16)
    for k in range(9):
        pltpu.sync_copy(idx_ref.at[k, wid, :], idxall.at[pl.ds(384 * k, 384)])

    def sub(s, _):
        nb = pl.multiple_of(base + s * 16, 16)
        pltpu.sync_copy(rep_ref.at[pl.ds(nb, 16), :], own_st)
        for k in range(8):
            idxb1[pl.ds(16 * k, 16)] = idxall[pl.ds(384 * k + 16 * s, 16)]
        idxb2[...] = idxall[pl.ds(384 * 8 + 16 * s, 16)]
        cp1 = pltpu.async_copy(rep_ref.at[idxb1], g1, sem1)
        cp2 = pltpu.async_copy(rep_ref.at[idxb2], g2, sem2)
        cp1.wait()
        cp2.wait()

        def node(i, _):
            own = [own_st[i, pl.ds(16 * j, 16)] for j in range(F // 16)]
            row = jnp.zeros((16,), jnp.float32)
            for k in range(9):
                if k < 8:
                    accv = own[0] * g1[16 * k + i, pl.ds(0, 16)]
                    for j in range(1, F // 16):
                        accv = accv + own[j] * g1[16 * k + i, pl.ds(16 * j, 16)]
                else:
                    accv = own[0] * g2[i, pl.ds(0, 16)]
                    for j in range(1, F // 16):
                        accv = accv + own[j] * g2[i, pl.ds(16 * j, 16)]
                row = jnp.where(lane == k, jnp.sum(accv), row)
            afst[i, :] = row
            return 0

        lax.fori_loop(0, 16, node, 0)
        pltpu.sync_copy(afst, af_hbm.at[pl.ds(nb, 16), :])
        return 0

    lax.fori_loop(0, NPT // 16, sub, 0)


def _proj0_body(x_ref, W0_ref, W1_ref, A0_ref, A1_ref,
                h0_ref, h1_ref, mx_ref):
    _proj_core(x_ref[...], (W0_ref, W1_ref), (A0_ref, A1_ref),
               (h0_ref, h1_ref), mx_ref)


def _proj1_body(x0_ref, x1_ref, beta_ref, W0_ref, W1_ref, A0_ref, A1_ref,
                h0_ref, h1_ref, mx_ref):
    x = beta_ref[0, 0] * x0_ref[...] + beta_ref[0, 1] * x1_ref[...]
    _proj_core(x, (W0_ref, W1_ref), (A0_ref, A1_ref),
               (h0_ref, h1_ref), mx_ref)


def _proj_core(x, W_refs, A_refs, h_refs, mx_ref):
    mxs = []
    for m in range(2):
        h = jnp.dot(x, W_refs[m][...], preferred_element_type=jnp.float32)
        ee = jnp.dot(h, A_refs[m][...], preferred_element_type=jnp.float32)
        h_refs[m][:, 0:F] = h
        h_refs[m][:, F:F + 32] = ee
        mxs.append(jnp.max(ee, axis=0).reshape(1, 1, 32))
    mx_ref[...] = jnp.concatenate(mxs, axis=2)


def _combine_body(a0_ref, d0_ref, h0_ref,
                  a1_ref, d1_ref, h1_ref,
                  m32_ref, X16_ref, Ws_ref, bs_ref, qs_ref,
                  z0_ref, z1_ref, wp_ref):
    gi = pl.program_id(0)
    rid = gi * RB + lax.broadcasted_iota(jnp.int32, (RB, 1), 0)
    msk = (rid < N).astype(jnp.float32)
    X16 = X16_ref[...]
    ws = []
    for m in range(2):
        h384 = (h0_ref, h1_ref)[m][...]
        h = h384[:, 0:F]
        el = h384[:, F:F + 16]
        er = h384[:, F + 16:F + 32]
        m16 = m32_ref[:, 16 * m:16 * m + 16]
        es = el + er
        es = jnp.maximum(es, 0.2 * es) - m16
        ees = jnp.exp(es)
        den = (d0_ref, d1_ref)[m][...] + ees
        numer = (a0_ref, a1_ref)[m][...] + h * jnp.dot(
            ees, X16, preferred_element_type=jnp.float32)
        z = numer / jnp.dot(den, X16, preferred_element_type=jnp.float32)
        z = jnp.where(z > 0, z, jnp.exp(z) - 1.0)
        ((z0_ref, z1_ref)[m])[...] = z
        t = jnp.tanh(jnp.dot(z, Ws_ref[...],
                             preferred_element_type=jnp.float32) + bs_ref[...])
        wv = jnp.sum(t * qs_ref[...], axis=1, keepdims=True)
        ws.append(jnp.sum(wv * msk).reshape(1, 1, 1))
    wp_ref[...] = jnp.concatenate(
        [ws[0], ws[1], jnp.zeros((1, 1, 126), jnp.float32)], axis=2)


def _rep_body(x0_ref, x1_ref, beta_ref, W_ref, rep_ref):
    x = beta_ref[0, 0] * x0_ref[...] + beta_ref[0, 1] * x1_ref[...]
    rep_ref[...] = jnp.dot(x, W_ref[...], preferred_element_type=jnp.float32)


def _loss_body(af_ref, out_ref):
    af = af_ref[...]
    rid = lax.broadcasted_iota(jnp.int32, (NPAD, 1), 0)
    msk = (rid < N).astype(jnp.float32)

    def sp(x):
        return jnp.maximum(x, 0.0) + jnp.log(1.0 + jnp.exp(-jnp.abs(x)))

    s_pos = jnp.sum(sp(-af[:, 0:4]) * msk)
    s_neg = jnp.sum(sp(af[:, 4:9]) * msk)
    out_ref[...] = jnp.reshape((s_pos + s_neg) / N, (1, 1))


def _f32(*shape):
    return jax.ShapeDtypeStruct(shape, jnp.float32)


def _row_spec(shape):
    return pl.BlockSpec(shape, lambda i: (i,) + (0,) * (len(shape) - 1))


def _fix_spec(shape):
    return pl.BlockSpec(shape, lambda i: (0,) * len(shape))


_SMEM_SPEC = pl.BlockSpec(memory_space=pltpu.SMEM)


def _proj_call(x_list, beta, W0, W1, A0, A1, din):
    outs = [_f32(NPAD, FE), _f32(NPAD, FE), _f32(GB, 1, 64)]
    out_specs = [_row_spec((RB, FE)), _row_spec((RB, FE)),
                 _row_spec((1, 1, 64))]
    w_specs = [_fix_spec((din, F)), _fix_spec((din, F)),
               _fix_spec((F, 32)), _fix_spec((F, 32))]
    if beta is None:
        return pl.pallas_call(
            _proj0_body, grid=(GB,),
            in_specs=[_row_spec((RB, din))] + w_specs,
            out_specs=out_specs, out_shape=outs,
        )(x_list[0], W0, W1, A0, A1)
    return pl.pallas_call(
        _proj1_body, grid=(GB,),
        in_specs=[_row_spec((RB, din)), _row_spec((RB, din)), _SMEM_SPEC]
        + w_specs,
        out_specs=out_specs, out_shape=outs,
    )(x_list[0], x_list[1], beta, W0, W1, A0, A1)


def _combine_call(a0, d0, h0, a1, d1, h1, m32, X16, Ws, bs, qs):
    return pl.pallas_call(
        _combine_body, grid=(GB,),
        in_specs=[
            _row_spec((RB, F)), _row_spec((RB, 16)), _row_spec((RB, FE)),
            _row_spec((RB, F)), _row_spec((RB, 16)), _row_spec((RB, FE)),
            _fix_spec((1, 32)), _fix_spec((16, F)), _fix_spec((F, 128)),
            _fix_spec((1, 128)), _fix_spec((1, 128)),
        ],
        out_specs=[_row_spec((RB, F)), _row_spec((RB, F)),
                   _row_spec((1, 1, 128))],
        out_shape=[_f32(NPAD, F), _f32(NPAD, F), _f32(GB, 1, 128)],
    )(a0, d0, h0, a1, d1, h1, m32, X16, Ws, bs, qs)


def _maxes(mx, m):
    blk = mx[:, 0, 32 * m:32 * m + 32]
    return jnp.max(blk[:, 0:16], axis=0) + jnp.max(blk[:, 16:32], axis=0)


def kernel(feats, mps_edge_index_0, mps_edge_index_1, train_edge_false_index,
           train_edge_f_index, train_edge_index, train_edge_ulu_index,
           train_edge_ullu_index, W_l0_m0, al_l0_m0, ar_l0_m0, W_l0_m1,
           al_l0_m1, ar_l0_m1, Ws_l0, bs_l0, qs_l0, W_l1_m0, al_l1_m0,
           ar_l1_m0, W_l1_m1, al_l1_m1, ar_l1_m1, Ws_l1, bs_l1, qs_l1, W_ed):
    f32 = jnp.float32
    xp = jnp.pad(feats, ((0, NPAD - N), (0, 0)))
    onehot = (jnp.arange(F)[:, None] // 64 == jnp.arange(16)[None, :]).astype(f32)

    def _A(al, ar):
        return jnp.concatenate([onehot * al.reshape(F)[:, None],
                                onehot * ar.reshape(F)[:, None]], axis=1)

    X16 = (jnp.arange(F)[None, :] // 64 == jnp.arange(16)[:, None]).astype(f32)
    A00, A01 = _A(al_l0_m0, ar_l0_m0), _A(al_l0_m1, ar_l0_m1)
    A10, A11 = _A(al_l1_m0, ar_l1_m0), _A(al_l1_m1, ar_l1_m1)

    bins, cnts = _bin_edges(mps_edge_index_0, mps_edge_index_1)

    zpair = None
    beta2d = None
    for l in range(2):
        W0, W1 = (W_l0_m0, W_l0_m1) if l == 0 else (W_l1_m0, W_l1_m1)
        A0, A1 = (A00, A01) if l == 0 else (A10, A11)
        Ws, bs, qs = (Ws_l0, bs_l0, qs_l0) if l == 0 else (Ws_l1, bs_l1, qs_l1)
        din = 128 if l == 0 else F
        x_list = [xp] if l == 0 else zpair
        h0, h1, mx = _proj_call(x_list, beta2d, W0, W1, A0, A1, din)
        M0 = _maxes(mx, 0)
        M1 = _maxes(mx, 1)
        er0f = h0[:, F + 16:F + 32].reshape(-1)
        er1f = h1[:, F + 16:F + 32].reshape(-1)
        acc0, den0 = _conv_g0(bins, cnts, h0, er0f, jnp.pad(M0, (0, 112)))
        acc1, den1 = _conv_g1(bins, cnts, h1, er1f, jnp.pad(M1, (0, 112)))
        den0 = den0.reshape(NPAD, 16)
        den1 = den1.reshape(NPAD, 16)
        m32 = jnp.concatenate([M0, M1]).reshape(1, 32)
        z0, z1, wp = _combine_call(acc0, den0, h0, acc1, den1, h1,
                                   m32, X16, Ws, bs.reshape(1, 128),
                                   qs.reshape(1, 128))
        beta = jax.nn.softmax(jnp.sum(wp[:, 0, 0:2], axis=0) / N)
        beta2d = beta.reshape(1, 2)
        zpair = [z0, z1]

    rep = pl.pallas_call(
        _rep_body, grid=(GB,),
        in_specs=[_row_spec((RB, F)), _row_spec((RB, F)), _SMEM_SPEC,
                  _fix_spec((F, F))],
        out_specs=_row_spec((RB, F)),
        out_shape=_f32(NPAD, F),
    )(zpair[0], zpair[1], beta2d, W_ed)

    idxs = jnp.concatenate([
        train_edge_f_index[None, :], train_edge_index[None, :],
        train_edge_ulu_index[None, :], train_edge_ullu_index[None, :],
        train_edge_false_index.T,
    ], axis=0)
    idx9 = jnp.pad(idxs, ((0, 0), (0, NPAD - N))).reshape(9, NW, NPT)
    idx9 = jnp.pad(idx9, ((0, 0), (0, 0), (0, 384 - NPT)))

    af = _loss_gather(rep, idx9)

    loss = pl.pallas_call(
        _loss_body,
        out_shape=_f32(1, 1),
    )(af)
    return jnp.reshape(loss, ())
